# baseline (speedup 1.0000x reference)
import functools

import jax
import jax.numpy as jnp
from jax import lax
from jax.experimental import pallas as pl
from jax.experimental.pallas import tpu as pltpu
from jax.experimental.pallas import tpu_sc as plsc

_NUM_CORES = 2
_NUM_SUBCORES = 16
_NW = _NUM_CORES * _NUM_SUBCORES
_LANES = 16


@functools.lru_cache(maxsize=None)
def _make_table_transpose(vocab: int, dim: int):
    blk_cols = 2048
    grid = (vocab + blk_cols - 1) // blk_cols

    def body(x_ref, o_ref):
        o_ref[...] = x_ref[...].T

    return pl.pallas_call(
        body,
        grid=(grid,),
        in_specs=[pl.BlockSpec((dim, blk_cols), lambda i: (0, i))],
        out_specs=pl.BlockSpec((blk_cols, dim), lambda i: (i, 0)),
        out_shape=jax.ShapeDtypeStruct((vocab, dim), jnp.float32),
    )


@functools.lru_cache(maxsize=None)
def _make_gather(batch: int, seq: int, vocab: int, dim: int):
    assert batch % _NW == 0
    b_per_w = batch // _NW
    n_blk = b_per_w // _LANES
    assert dim == 32
    assert seq % 2 == 0

    mesh = plsc.VectorSubcoreMesh(core_axis_name="c", subcore_axis_name="s")

    @functools.partial(
        pl.kernel,
        mesh=mesh,
        compiler_params=pltpu.CompilerParams(
            use_tc_tiling_on_sc=False, needs_layout_passes=False
        ),
        out_type=jax.ShapeDtypeStruct((seq, dim, batch), jnp.float32),
        scratch_types=[
            pltpu.VMEM((b_per_w,), jnp.int32),
            pltpu.VMEM((b_per_w,), jnp.int32),
            pltpu.VMEM((b_per_w, dim), jnp.float32),
            pltpu.VMEM((b_per_w, dim), jnp.float32),
            pltpu.VMEM((dim, b_per_w + 1), jnp.float32),
            pltpu.VMEM((dim, b_per_w + 1), jnp.float32),
            pltpu.SemaphoreType.DMA,
            pltpu.SemaphoreType.DMA,
            pltpu.SemaphoreType.DMA,
            pltpu.SemaphoreType.DMA,
        ],
    )
    def gather_kernel(
        table_hbm, idx_hbm, out_hbm,
        idx0, idx1, rows0, rows1, tv0, tv1, gsem0, gsem1, wsem0, wsem1,
    ):
        idx_bufs = (idx0, idx1)
        rows = (rows0, rows1)
        tvs = (tv0, tv1)
        gsems = (gsem0, gsem1)
        wsems = (wsem0, wsem1)

        wid = lax.axis_index("s") * _NUM_CORES + lax.axis_index("c")
        b0 = wid * b_per_w
        iota = lax.iota(jnp.int32, _LANES)

        def start(l, k):
            pltpu.sync_copy(idx_hbm.at[pl.ds(l * batch + b0, b_per_w)], idx_bufs[k])
            pltpu.async_copy(table_hbm.at[idx_bufs[k]], rows[k], gsems[k])

        start(0, 0)

        def body(i, carry):
            for k in range(2):
                l = 2 * i + k
                if k == 0:
                    start(l + 1, 1)
                else:
                    @pl.when(i < seq // 2 - 1)
                    def _():
                        start(l + 1, 0)

                pltpu.make_async_copy(
                    table_hbm.at[idx_bufs[k]], rows[k], gsems[k]
                ).wait()

                @pl.when(i >= 1)
                def _():
                    pltpu.make_async_copy(
                        tvs[k].at[:, pl.ds(0, b_per_w)],
                        out_hbm.at[l, :, pl.ds(b0, b_per_w)],
                        wsems[k],
                    ).wait()

                def tr(jb, c):
                    for rr in range(8):
                        r = jb * 8 + rr
                        rsplat = jnp.full((_LANES,), r, jnp.int32)
                        for d0 in range(0, dim, _LANES):
                            v = rows[k][r, pl.ds(d0, _LANES)]
                            plsc.store_scatter(tvs[k], [iota + d0, rsplat], v)
                    return c

                lax.fori_loop(0, b_per_w // 8, tr, 0)

                pltpu.async_copy(
                    tvs[k].at[:, pl.ds(0, b_per_w)],
                    out_hbm.at[l, :, pl.ds(b0, b_per_w)],
                    wsems[k],
                )
            return carry

        lax.fori_loop(0, seq // 2, body, 0)

        for k in range(2):
            pltpu.make_async_copy(
                tvs[k].at[:, pl.ds(0, b_per_w)],
                out_hbm.at[0, :, pl.ds(b0, b_per_w)],
                wsems[k],
            ).wait()

    return gather_kernel


def kernel(inputs, table):
    b, l = inputs.shape
    vocab, dim = table.shape
    flat_idx = inputs.T.reshape(b * l)
    tbl = _make_table_transpose(vocab, dim)(table.T)
    out = _make_gather(b, l, vocab, dim)(tbl, flat_idx)
    return out.transpose(2, 0, 1)


# speedup vs baseline: 1.3237x; 1.3237x over previous
import functools

import jax
import jax.numpy as jnp
from jax import lax
from jax.experimental import pallas as pl
from jax.experimental.pallas import tpu as pltpu
from jax.experimental.pallas import tpu_sc as plsc

_NUM_CORES = 2
_NUM_SUBCORES = 16
_NW = _NUM_CORES * _NUM_SUBCORES
_LANES = 16


@functools.lru_cache(maxsize=None)
def _make_gather(batch: int, seq: int, vocab: int, dim: int):
    assert batch % _NW == 0
    b_per_w = batch // _NW
    n_blk = b_per_w // _LANES
    assert dim == 32
    assert seq % 2 == 0
    halves = dim // _LANES

    mesh = plsc.VectorSubcoreMesh(core_axis_name="c", subcore_axis_name="s")

    @functools.partial(
        pl.kernel,
        mesh=mesh,
        compiler_params=pltpu.CompilerParams(
            use_tc_tiling_on_sc=False, needs_layout_passes=False
        ),
        out_type=jax.ShapeDtypeStruct((seq, dim, batch), jnp.float32),
        scratch_types=[
            pltpu.VMEM((b_per_w,), jnp.int32),
            pltpu.VMEM((b_per_w,), jnp.int32),
            pltpu.VMEM((halves * b_per_w,), jnp.int32),
            pltpu.VMEM((halves * b_per_w,), jnp.int32),
            pltpu.VMEM((halves * b_per_w, _LANES), jnp.float32),
            pltpu.VMEM((halves * b_per_w, _LANES), jnp.float32),
            pltpu.VMEM((dim, b_per_w + 1), jnp.float32),
            pltpu.VMEM((dim, b_per_w + 1), jnp.float32),
            pltpu.SemaphoreType.DMA,
            pltpu.SemaphoreType.DMA,
            pltpu.SemaphoreType.DMA,
            pltpu.SemaphoreType.DMA,
        ],
    )
    def gather_kernel(
        table_hbm, idx_hbm, out_hbm,
        idx0, idx1, didx0, didx1, rows0, rows1, tv0, tv1,
        gsem0, gsem1, wsem0, wsem1,
    ):
        idx_bufs = (idx0, idx1)
        didx_bufs = (didx0, didx1)
        rows = (rows0, rows1)
        tvs = (tv0, tv1)
        gsems = (gsem0, gsem1)
        wsems = (wsem0, wsem1)

        wid = lax.axis_index("s") * _NUM_CORES + lax.axis_index("c")
        b0 = wid * b_per_w
        iota = lax.iota(jnp.int32, _LANES)

        def start(l, k):
            pltpu.sync_copy(idx_hbm.at[pl.ds(l * batch + b0, b_per_w)], idx_bufs[k])

            def bld(jb, c):
                v = idx_bufs[k][pl.ds(jb * _LANES, _LANES)] * halves
                for h in range(halves):
                    didx_bufs[k][pl.ds(h * b_per_w + jb * _LANES, _LANES)] = v + h
                return c

            lax.fori_loop(0, n_blk, bld, 0, unroll=4)
            pltpu.async_copy(table_hbm.at[didx_bufs[k]], rows[k], gsems[k])

        start(0, 0)

        def body(i, carry):
            for k in range(2):
                l = 2 * i + k
                if k == 0:
                    start(l + 1, 1)
                else:
                    @pl.when(i < seq // 2 - 1)
                    def _():
                        start(l + 1, 0)

                pltpu.make_async_copy(
                    table_hbm.at[didx_bufs[k]], rows[k], gsems[k]
                ).wait()

                @pl.when(i >= 1)
                def _():
                    pltpu.make_async_copy(
                        tvs[k].at[:, pl.ds(0, b_per_w)],
                        out_hbm.at[l, :, pl.ds(b0, b_per_w)],
                        wsems[k],
                    ).wait()

                def tr(jb, c):
                    for rr in range(8):
                        r = jb * 8 + rr
                        rsplat = jnp.full((_LANES,), r, jnp.int32)
                        for h in range(halves):
                            v = rows[k][h * b_per_w + r, :]
                            plsc.store_scatter(
                                tvs[k], [iota + h * _LANES, rsplat], v
                            )
                    return c

                lax.fori_loop(0, b_per_w // 8, tr, 0)

                pltpu.async_copy(
                    tvs[k].at[:, pl.ds(0, b_per_w)],
                    out_hbm.at[l, :, pl.ds(b0, b_per_w)],
                    wsems[k],
                )
            return carry

        lax.fori_loop(0, seq // 2, body, 0)

        for k in range(2):
            pltpu.make_async_copy(
                tvs[k].at[:, pl.ds(0, b_per_w)],
                out_hbm.at[0, :, pl.ds(b0, b_per_w)],
                wsems[k],
            ).wait()

    return gather_kernel


def kernel(inputs, table):
    b, l = inputs.shape
    vocab, dim = table.shape
    flat_idx = inputs.T.reshape(b * l)
    tbl16 = table.reshape(vocab * dim // _LANES, _LANES)
    out = _make_gather(b, l, vocab, dim)(tbl16, flat_idx)
    return out.transpose(2, 0, 1)


# speedup vs baseline: 1.3688x; 1.0341x over previous
import functools

import jax
import jax.numpy as jnp
from jax import lax
from jax.experimental import pallas as pl
from jax.experimental.pallas import tpu as pltpu
from jax.experimental.pallas import tpu_sc as plsc

_NUM_CORES = 2
_NUM_SUBCORES = 16
_NW = _NUM_CORES * _NUM_SUBCORES
_LANES = 16


@functools.lru_cache(maxsize=None)
def _make_gather(batch: int, seq: int, vocab: int, dim: int):
    assert batch % _NW == 0
    b_per_w = batch // _NW
    n_blk = b_per_w // _LANES
    assert dim == 32
    assert seq % 2 == 0
    halves = dim // _LANES

    mesh = plsc.VectorSubcoreMesh(core_axis_name="c", subcore_axis_name="s")

    @functools.partial(
        pl.kernel,
        mesh=mesh,
        compiler_params=pltpu.CompilerParams(
            use_tc_tiling_on_sc=False, needs_layout_passes=False
        ),
        out_type=jax.ShapeDtypeStruct((seq, dim, batch), jnp.float32),
        scratch_types=[
            pltpu.VMEM((seq, b_per_w), jnp.int32),
            pltpu.VMEM((halves * b_per_w,), jnp.int32),
            pltpu.VMEM((halves * b_per_w,), jnp.int32),
            pltpu.VMEM((halves * b_per_w, _LANES), jnp.float32),
            pltpu.VMEM((halves * b_per_w, _LANES), jnp.float32),
            pltpu.VMEM((dim, b_per_w + 1), jnp.float32),
            pltpu.VMEM((dim, b_per_w + 1), jnp.float32),
            pltpu.SemaphoreType.DMA,
            pltpu.SemaphoreType.DMA,
            pltpu.SemaphoreType.DMA,
            pltpu.SemaphoreType.DMA,
        ],
    )
    def gather_kernel(
        table_hbm, idx_hbm, out_hbm,
        idx_all, didx0, didx1, rows0, rows1, tv0, tv1,
        gsem0, gsem1, wsem0, wsem1,
    ):
        didx_bufs = (didx0, didx1)
        rows = (rows0, rows1)
        tvs = (tv0, tv1)
        gsems = (gsem0, gsem1)
        wsems = (wsem0, wsem1)

        wid = lax.axis_index("s") * _NUM_CORES + lax.axis_index("c")
        b0 = wid * b_per_w

        pltpu.sync_copy(idx_hbm.at[:, pl.ds(b0, b_per_w)], idx_all)

        def start(l, k):
            def bld(jb, c):
                v = idx_all[l, pl.ds(jb * _LANES, _LANES)] * halves
                for h in range(halves):
                    didx_bufs[k][pl.ds(h * b_per_w + jb * _LANES, _LANES)] = v + h
                return c

            lax.fori_loop(0, n_blk, bld, 0, unroll=4)
            pltpu.async_copy(table_hbm.at[didx_bufs[k]], rows[k], gsems[k])

        start(0, 0)

        def body(i, carry):
            for k in range(2):
                l = 2 * i + k
                if k == 0:
                    start(l + 1, 1)
                else:
                    @pl.when(i < seq // 2 - 1)
                    def _():
                        start(l + 1, 0)

                pltpu.make_async_copy(
                    table_hbm.at[didx_bufs[k]], rows[k], gsems[k]
                ).wait()

                @pl.when(i >= 1)
                def _():
                    pltpu.make_async_copy(
                        tvs[k].at[:, pl.ds(0, b_per_w)],
                        out_hbm.at[l, :, pl.ds(b0, b_per_w)],
                        wsems[k],
                    ).wait()

                iota = lax.iota(jnp.int32, _LANES)

                def tr(jb, c):
                    for rr in range(8):
                        r = jb * 8 + rr
                        rsplat = jnp.full((_LANES,), r, jnp.int32)
                        for h in range(halves):
                            v = rows[k][h * b_per_w + r, :]
                            plsc.store_scatter(
                                tvs[k], [iota + h * _LANES, rsplat], v
                            )
                    return c

                lax.fori_loop(0, b_per_w // 8, tr, 0)

                pltpu.async_copy(
                    tvs[k].at[:, pl.ds(0, b_per_w)],
                    out_hbm.at[l, :, pl.ds(b0, b_per_w)],
                    wsems[k],
                )
            return carry

        lax.fori_loop(0, seq // 2, body, 0)

        for k in range(2):
            pltpu.make_async_copy(
                tvs[k].at[:, pl.ds(0, b_per_w)],
                out_hbm.at[0, :, pl.ds(b0, b_per_w)],
                wsems[k],
            ).wait()

    return gather_kernel


def kernel(inputs, table):
    b, l = inputs.shape
    vocab, dim = table.shape
    idx_lmajor = inputs.T
    tbl16 = table.reshape(vocab * dim // _LANES, _LANES)
    out = _make_gather(b, l, vocab, dim)(tbl16, idx_lmajor)
    return out.transpose(2, 0, 1)


# speedup vs baseline: 1.3703x; 1.0011x over previous
import functools

import jax
import jax.numpy as jnp
from jax import lax
from jax.experimental import pallas as pl
from jax.experimental.pallas import tpu as pltpu
from jax.experimental.pallas import tpu_sc as plsc

_NUM_CORES = 2
_NUM_SUBCORES = 16
_NW = _NUM_CORES * _NUM_SUBCORES
_LANES = 16


@functools.lru_cache(maxsize=None)
def _make_gather(batch: int, seq: int, vocab: int, dim: int):
    assert batch % _NW == 0
    b_per_w = batch // _NW
    n_blk = b_per_w // _LANES
    assert dim == 32
    assert seq % 2 == 0
    halves = dim // _LANES

    mesh = plsc.VectorSubcoreMesh(core_axis_name="c", subcore_axis_name="s")

    @functools.partial(
        pl.kernel,
        mesh=mesh,
        compiler_params=pltpu.CompilerParams(
            use_tc_tiling_on_sc=False, needs_layout_passes=False
        ),
        out_type=jax.ShapeDtypeStruct((seq, dim, batch), jnp.float32),
        scratch_types=[
            pltpu.VMEM((seq, b_per_w), jnp.int32),
            pltpu.VMEM((halves * b_per_w,), jnp.int32),
            pltpu.VMEM((halves * b_per_w,), jnp.int32),
            pltpu.VMEM((halves * b_per_w, _LANES), jnp.float32),
            pltpu.VMEM((halves * b_per_w, _LANES), jnp.float32),
            pltpu.VMEM((dim, b_per_w + 1), jnp.float32),
            pltpu.VMEM((dim, b_per_w + 1), jnp.float32),
            pltpu.SemaphoreType.DMA,
            pltpu.SemaphoreType.DMA,
            pltpu.SemaphoreType.DMA,
            pltpu.SemaphoreType.DMA,
        ],
    )
    def gather_kernel(
        table_hbm, idx_hbm, out_hbm,
        idx_all, didx0, didx1, rows0, rows1, tv0, tv1,
        gsem0, gsem1, wsem0, wsem1,
    ):
        didx_bufs = (didx0, didx1)
        rows = (rows0, rows1)
        tvs = (tv0, tv1)
        gsems = (gsem0, gsem1)
        wsems = (wsem0, wsem1)

        wid = lax.axis_index("s") * _NUM_CORES + lax.axis_index("c")
        b0 = wid * b_per_w

        pltpu.sync_copy(idx_hbm.at[:, pl.ds(b0, b_per_w)], idx_all)

        def start(l, k):
            def bld(jb, c):
                v = idx_all[l, pl.ds(jb * _LANES, _LANES)] * halves
                for h in range(halves):
                    didx_bufs[k][pl.ds(h * b_per_w + jb * _LANES, _LANES)] = v + h
                return c

            lax.fori_loop(0, n_blk, bld, 0, unroll=4)
            pltpu.async_copy(table_hbm.at[didx_bufs[k]], rows[k], gsems[k])

        start(0, 0)

        def body(i, carry):
            for k in range(2):
                l = 2 * i + k
                if k == 0:
                    start(l + 1, 1)
                else:
                    @pl.when(i < seq // 2 - 1)
                    def _():
                        start(l + 1, 0)

                pltpu.make_async_copy(
                    table_hbm.at[didx_bufs[k]], rows[k], gsems[k]
                ).wait()

                @pl.when(i >= 1)
                def _():
                    pltpu.make_async_copy(
                        tvs[k].at[:, pl.ds(0, b_per_w)],
                        out_hbm.at[l, :, pl.ds(b0, b_per_w)],
                        wsems[k],
                    ).wait()

                iota = lax.iota(jnp.int32, _LANES)

                def tr(jb, c):
                    for rr in range(8):
                        r = jb * 8 + rr
                        rsplat = jnp.full((_LANES,), r, jnp.int32)
                        for h in range(halves):
                            v = rows[k][h * b_per_w + r, :]
                            plsc.store_scatter(
                                tvs[k], [iota + h * _LANES, rsplat], v
                            )
                    return c

                lax.fori_loop(0, b_per_w // 8, tr, 0, unroll=2)

                pltpu.async_copy(
                    tvs[k].at[:, pl.ds(0, b_per_w)],
                    out_hbm.at[l, :, pl.ds(b0, b_per_w)],
                    wsems[k],
                )
            return carry

        lax.fori_loop(0, seq // 2, body, 0)

        for k in range(2):
            pltpu.make_async_copy(
                tvs[k].at[:, pl.ds(0, b_per_w)],
                out_hbm.at[0, :, pl.ds(b0, b_per_w)],
                wsems[k],
            ).wait()

    return gather_kernel


def kernel(inputs, table):
    b, l = inputs.shape
    vocab, dim = table.shape
    idx_lmajor = inputs.T
    tbl16 = table.reshape(vocab * dim // _LANES, _LANES)
    out = _make_gather(b, l, vocab, dim)(tbl16, idx_lmajor)
    return out.transpose(2, 0, 1)
